# baseline (speedup 1.0000x reference)
import functools

import jax
import jax.numpy as jnp
from jax import lax
from jax.experimental import pallas as pl
from jax.experimental.pallas import tpu as pltpu

N_NODES = 10000
N_EDGES = 160000
D_NODE = 128
D_EDGE = 64

_EB = 2000
_NB = 2000


def _full(shape):
    return pl.BlockSpec(shape, lambda i: tuple(0 for _ in shape))


def _enc_body(x_ref, w0, b0, w1, b1, o_ref):
    h = jnp.maximum(x_ref[...] @ w0[...] + b0[...], 0.0)
    o_ref[...] = jnp.maximum(h @ w1[...] + b1[...], 0.0)


def _encoder(x, w0, b0, w1, b1, block):
    n = x.shape[0]
    d_in, d_h = w0.shape
    d_out = w1.shape[1]
    return pl.pallas_call(
        _enc_body,
        grid=(n // block,),
        in_specs=[
            pl.BlockSpec((block, d_in), lambda i: (i, 0)),
            _full((d_in, d_h)), _full((1, d_h)),
            _full((d_h, d_out)), _full((1, d_out)),
        ],
        out_specs=pl.BlockSpec((block, d_out), lambda i: (i, 0)),
        out_shape=jax.ShapeDtypeStruct((n, d_out), jnp.float32),
        compiler_params=pltpu.CompilerParams(
            dimension_semantics=("arbitrary",)),
    )(x, w0, b0, w1, b1)


def _edge_body(xr, xc, ie, le,
               w1r, w1c, w1i, w1l, b1, w2, b2,
               wfoc, wfoe, bfo1, wfo2, bfo2,
               wfic, wfie, bfi1, wfi2, bfi2,
               wc1, bc1, wc2, bc2,
               le_o, fo_o, fi_o, cls_o):
    h = (xr[...] @ w1r[...] + xc[...] @ w1c[...]
         + ie[...] @ w1i[...] + le[...] @ w1l[...] + b1[...])
    h = jnp.maximum(h, 0.0)
    le_new = jnp.maximum(h @ w2[...] + b2[...], 0.0)
    le_o[...] = le_new

    xc_v = xc[...]
    ho = jnp.maximum(xc_v @ wfoc[...] + le_new @ wfoe[...] + bfo1[...], 0.0)
    fo_o[...] = jnp.maximum(ho @ wfo2[...] + bfo2[...], 0.0)
    hi = jnp.maximum(xc_v @ wfic[...] + le_new @ wfie[...] + bfi1[...], 0.0)
    fi_o[...] = jnp.maximum(hi @ wfi2[...] + bfi2[...], 0.0)

    hc = jnp.maximum(le_new @ wc1[...] + bc1[...], 0.0)
    cls_o[...] = hc @ wc2[...] + bc2[...]


def _edge_step(xr, xc, ie, le, ew):
    grid = (N_EDGES // _EB,)
    row_spec = lambda d: pl.BlockSpec((_EB, d), lambda i: (i, 0))
    in_specs = [row_spec(D_NODE), row_spec(D_NODE), row_spec(D_EDGE),
                row_spec(D_EDGE)] + [_full(w.shape) for w in ew]
    out_specs = [row_spec(D_EDGE), row_spec(D_NODE), row_spec(D_NODE),
                 row_spec(1)]
    out_shape = [
        jax.ShapeDtypeStruct((N_EDGES, D_EDGE), jnp.float32),
        jax.ShapeDtypeStruct((N_EDGES, D_NODE), jnp.float32),
        jax.ShapeDtypeStruct((N_EDGES, D_NODE), jnp.float32),
        jax.ShapeDtypeStruct((N_EDGES, 1), jnp.float32),
    ]
    return pl.pallas_call(
        _edge_body,
        grid=grid,
        in_specs=in_specs,
        out_specs=out_specs,
        out_shape=out_shape,
        compiler_params=pltpu.CompilerParams(
            dimension_semantics=("arbitrary",)),
    )(xr, xc, ie, le, *ew)


def _node_body(fi, fo, wni, wno, bn, o_ref):
    o_ref[...] = jnp.maximum(
        fi[...] @ wni[...] + fo[...] @ wno[...] + bn[...], 0.0)


def _node_step(fi, fo, wni, wno, bn):
    return pl.pallas_call(
        _node_body,
        grid=(N_NODES // _NB,),
        in_specs=[
            pl.BlockSpec((_NB, D_NODE), lambda i: (i, 0)),
            pl.BlockSpec((_NB, D_NODE), lambda i: (i, 0)),
            _full((D_NODE, D_NODE)), _full((D_NODE, D_NODE)),
            _full((1, D_NODE)),
        ],
        out_specs=pl.BlockSpec((_NB, D_NODE), lambda i: (i, 0)),
        out_shape=jax.ShapeDtypeStruct((N_NODES, D_NODE), jnp.float32),
        compiler_params=pltpu.CompilerParams(
            dimension_semantics=("arbitrary",)),
    )(fi, fo, wni, wno, bn)


def _gather_rows(table, idx):
    return jnp.take(table, idx, axis=0)


def _segment_sum(vals, idx):
    return jax.ops.segment_sum(vals, idx, num_segments=N_NODES + 8)[:N_NODES]


def kernel(x, edge_attr, params, edge_index):
    row = edge_index[0]
    col = edge_index[1]
    idx_out = jnp.where(row < col, row, N_NODES).astype(jnp.int32)
    idx_in = jnp.where(row > col, row, N_NODES).astype(jnp.int32)

    p = params

    def wb(layer):
        w, b = layer
        return w, b.reshape(1, -1)

    en0w, en0b = wb(p['enc_node'][0])
    en1w, en1b = wb(p['enc_node'][1])
    ee0w, ee0b = wb(p['enc_edge'][0])
    ee1w, ee1b = wb(p['enc_edge'][1])

    em0w, em0b = wb(p['edge_model'][0])
    em1w, em1b = wb(p['edge_model'][1])
    w1r, w1c = em0w[0:128], em0w[128:256]
    w1i, w1l = em0w[256:320], em0w[320:384]

    fo0w, fo0b = wb(p['flow_out_mlp'][0])
    fo1w, fo1b = wb(p['flow_out_mlp'][1])
    wfoc, wfoe = fo0w[0:128], fo0w[128:192]
    fi0w, fi0b = wb(p['flow_in_mlp'][0])
    fi1w, fi1b = wb(p['flow_in_mlp'][1])
    wfic, wfie = fi0w[0:128], fi0w[128:192]

    nw, nb = wb(p['node_mlp'][0])
    wni, wno = nw[0:128], nw[128:256]

    c0w, c0b = wb(p['classifier_edge'][0])
    c1w, c1b = wb(p['classifier_edge'][1])

    ew = (w1r, w1c, w1i, w1l, em0b, em1w, em1b,
          wfoc, wfoe, fo0b, fo1w, fo1b,
          wfic, wfie, fi0b, fi1w, fi1b,
          c0w, c0b, c1w, c1b)

    ln = _encoder(x, en0w, en0b, en1w, en1b, 2000)
    init_edge = _encoder(edge_attr, ee0w, ee0b, ee1w, ee1b, 8000)
    le = init_edge

    outputs = []
    for step in range(1, 5):
        xr = _gather_rows(ln, row)
        xc = _gather_rows(ln, col)
        le, fo, fi, cls = _edge_step(xr, xc, init_edge, le, ew)
        fo_n = _segment_sum(fo, idx_out)
        fi_n = _segment_sum(fi, idx_in)
        ln = _node_step(fi_n, fo_n, wni, wno, nb)
        if step >= 3:
            outputs.append(cls)
    return jnp.stack(outputs)
